# Initial kernel scaffold; baseline (speedup 1.0000x reference)
import jax
import jax.numpy as jnp
from jax.experimental import pallas as pl


def kernel(input_y, sigma):
    raise NotImplementedError("write your pallas kernel here")


# speedup vs baseline: 1.0007x; 1.0007x over previous
import jax
import jax.numpy as jnp
import numpy as np
from functools import partial
from jax.experimental import pallas as pl

_P1, _P2, _M1, _M2, _WIN, _STEP = 7, 7, 18, 55, 37, 4


def _unfold(x, p):
    N, C, H, W = x.shape
    Hc, Wc = H - p + 1, W - p + 1
    cols = jnp.stack([x[:, :, di:di + Hc, dj:dj + Wc] for di in range(p) for dj in range(p)], axis=2)
    return cols.reshape(N, C * p * p, Hc * Wc)


def _fold(X, H, W, p, C):
    N = X.shape[0]
    Hc, Wc = H - p + 1, W - p + 1
    Xr = X.reshape(N, C, p, p, Hc, Wc)
    out = jnp.zeros((N, C, H, W), X.dtype)
    for di in range(p):
        for dj in range(p):
            out = out.at[:, :, di:di + Hc, dj:dj + Wc].add(Xr[:, :, di, dj])
    return out


def _align_corners(x, s, value=0):
    N, C, H, W = x.shape
    if s == 1 or (H % s == 1 and W % s == 1):
        return x
    i_pad, j_pad = (s - H % s + 1) % s, (s - W % s + 1) % s
    xp = jnp.pad(x, ((0, 0), (0, 0), (0, i_pad), (0, j_pad)), constant_values=value)
    xp = xp.at[:, :, -1:, :W:s].set(x[:, :, -1:, ::s])
    xp = xp.at[:, :, :H:s, -1:].set(x[:, :, ::s, -1:])
    xp = xp.at[:, :, -1:, -1:].set(x[:, :, -1:, -1:])
    if i_pad > 0:
        xp = xp.at[:, :, H - 1:H, :W:s].set(value)
    if j_pad > 0:
        xp = xp.at[:, :, :H:s, W - 1:W].set(value)
    if i_pad > 0 and j_pad > 0:
        xp = xp.at[:, :, H - 1:H, W - 1:W].set(value)
    return xp


@partial(jax.jit, static_argnums=(2, 3))
def _block_dist(x_pad, x_center, w, s):
    N, F, Ip, Jp = x_pad.shape
    v = w // 2
    I, J = Ip - 2 * v, Jp - 2 * v
    Ic, Jc = x_center.shape[2], x_center.shape[3]

    def body(idx, acc):
        i, j = idx // w, idx % w
        sl = jax.lax.dynamic_slice(x_pad, (0, 0, i, j), (N, F, I, J))[:, :, ::s, ::s]
        return acc.at[:, idx].set(jnp.mean((sl - x_center) ** 2, axis=1))

    return jax.lax.fori_loop(0, w * w, body, jnp.zeros((N, w * w, Ic, Jc), x_pad.dtype))


def _block_matching(input_x, m, p):
    input_x = jax.lax.stop_gradient(input_x)
    N, C, H, W = input_x.shape
    w, s = _WIN, _STEP
    v = w // 2
    Hc, Wc = H - p + 1, W - p + 1
    x_patches = _unfold(input_x, p).reshape(N, C * p * p, Hc, Wc)
    x_patches = _align_corners(x_patches, s, value=float('inf'))
    x_pad = jnp.pad(x_patches, ((0, 0), (0, 0), (v, v), (v, v)), constant_values=float('inf'))
    x_center = x_patches[:, :, ::s, ::s]
    x_dist = _block_dist(x_pad, x_center, w, s)
    x_dist = x_dist.at[:, v * w + v].set(-float('inf'))
    xd = jnp.moveaxis(x_dist, 1, -1)
    _, ind = jax.lax.top_k(-xd, m)
    indices = jnp.moveaxis(ind, -1, 1)
    ind_row = indices // w - v
    ind_col = indices % w - v
    pr = jnp.broadcast_to(jnp.arange(H + w - p).reshape(1, 1, H + w - p, 1), (N, m, H + w - p, W + w - p))[:, :, v:Hc + v, v:Wc + v]
    pc = jnp.broadcast_to(jnp.arange(W + w - p).reshape(1, 1, 1, W + w - p), (N, m, H + w - p, W + w - p))[:, :, v:Hc + v, v:Wc + v]
    pr = _align_corners(pr, s, value=0)[:, :, ::s, ::s]
    pc = _align_corners(pc, s, value=0)[:, :, ::s, ::s]
    indices_row = jnp.minimum(ind_row + pr - v, H - p)
    indices_col = jnp.minimum(ind_col + pc - v, W - p)
    idxf = indices_row * (W - p + 1) + indices_col
    idxf = idxf.reshape(N, m, -1)
    idxf = jnp.transpose(idxf, (0, 2, 1)).reshape(N, -1)
    return idxf


def _group_patches(input_y, indices, m, n, p):
    N = input_y.shape[0]
    unfold_y = _unfold(input_y, p)
    Y = jax.vmap(lambda u, ix: u[:, ix])(unfold_y, indices)
    return jnp.transpose(Y, (0, 2, 1)).reshape(N, -1, m, n)


def _cho_solve(L, B):
    y = jax.lax.linalg.triangular_solve(L, B, left_side=True, lower=True)
    return jax.lax.linalg.triangular_solve(L, y, left_side=True, lower=True, transpose_a=True)


def _denoise1(Y, sigma):
    N, B, m, n = Y.shape
    YtY = Y @ jnp.swapaxes(Y, 2, 3)
    Im = jnp.broadcast_to(jnp.eye(m, dtype=Y.dtype), (N, B, m, m))
    L = jnp.linalg.cholesky(YtY)
    theta = _cho_solve(L, YtY - n * (sigma.reshape(()) ** 2) * Im)
    X_hat = theta @ Y
    weights = 1.0 / jnp.clip(jnp.sum(theta ** 2, axis=3, keepdims=True), 1.0 / m, 1.0)
    return X_hat, weights


def _denoise2(Y, X, sigma):
    N, B, m, n = Y.shape
    XtX = X @ jnp.swapaxes(X, 2, 3)
    Im = jnp.broadcast_to(jnp.eye(m, dtype=Y.dtype), (N, B, m, m))
    L = jnp.linalg.cholesky(XtX + n * (sigma.reshape(()) ** 2) * Im)
    theta = _cho_solve(L, XtX)
    X_hat = theta @ Y
    weights = 1.0 / jnp.clip(jnp.sum(theta ** 2, axis=3, keepdims=True), 1.0 / m, 1.0)
    return X_hat, weights


def _aggregation(X_hat, weights, indices, size, p):
    N, C, H, W = size
    n = C * p * p
    Ltot = (H - p + 1) * (W - p + 1)
    X_hat = X_hat * weights
    Xp = jnp.transpose(X_hat, (0, 3, 1, 2)).reshape(N, n, -1)
    wts = jnp.broadcast_to(weights.reshape(N, 1, -1), Xp.shape)
    scat = lambda ix, vals: jnp.zeros((n, Ltot), vals.dtype).at[:, ix].add(vals)
    X_sum = jax.vmap(scat)(indices, Xp)
    divisor = jax.vmap(scat)(indices, wts)
    return _pallas_div(_fold(X_sum, H, W, p, C), _fold(divisor, H, W, p, C))


def _div_body(a_ref, b_ref, o_ref):
    o_ref[...] = a_ref[...] / b_ref[...]


def _pallas_div(a, b):
    return pl.pallas_call(
        _div_body,
        out_shape=jax.ShapeDtypeStruct(a.shape, a.dtype),
    )(a, b)


def _step1(input_y, sigma):
    p, m = _P1, _M1
    y_block = jnp.mean(input_y, axis=1, keepdims=True)
    indices = _block_matching(y_block, m, p)
    C = input_y.shape[1]
    Y = _group_patches(input_y, indices, m, C * p * p, p)
    X_hat, weights = _denoise1(Y, sigma)
    return _aggregation(X_hat, weights, indices, input_y.shape, p)


def _step2(input_y, input_x, sigma):
    p, m = _P2, _M2
    x_block = jnp.mean(input_x, axis=1, keepdims=True)
    indices = _block_matching(x_block, m, p)
    C = input_y.shape[1]
    Y = _group_patches(input_y, indices, m, C * p * p, p)
    X = _group_patches(input_x, indices, m, C * p * p, p)
    X_hat, weights = _denoise2(Y, X, sigma)
    return _aggregation(X_hat, weights, indices, input_y.shape, p)


def kernel(input_y, sigma):
    den1 = _step1(input_y, sigma)
    den2 = _step2(input_y, den1, sigma)
    return den2


# speedup vs baseline: 8.8091x; 8.8029x over previous
import jax
import jax.numpy as jnp
import numpy as np
from functools import partial
from jax.experimental import pallas as pl
from jax.experimental.pallas import tpu as pltpu

_P, _M1, _M2, _WIN, _STEP = 7, 18, 55, 37, 4
_V = _WIN // 2
_H = _WIN_SQ = 37 * 37
_HPAD = 1376
_SELF = _V * _WIN + _V
_L1D = 218
_NC = 56
_LC = _NC * _NC


def _unfold(x, p):
    N, C, H, W = x.shape
    Hc, Wc = H - p + 1, W - p + 1
    cols = jnp.stack([x[:, :, di:di + Hc, dj:dj + Wc] for di in range(p) for dj in range(p)], axis=2)
    return cols.reshape(N, C * p * p, Hc * Wc)


def _fold(X, H, W, p, C):
    N = X.shape[0]
    Hc, Wc = H - p + 1, W - p + 1
    Xr = X.reshape(N, C, p, p, Hc, Wc)
    out = jnp.zeros((N, C, H, W), X.dtype)
    for di in range(p):
        for dj in range(p):
            out = out.at[:, :, di:di + Hc, dj:dj + Wc].add(Xr[:, :, di, dj])
    return out


def _align_corners(x, s, value=0):
    N, C, H, W = x.shape
    if s == 1 or (H % s == 1 and W % s == 1):
        return x
    i_pad, j_pad = (s - H % s + 1) % s, (s - W % s + 1) % s
    xp = jnp.pad(x, ((0, 0), (0, 0), (0, i_pad), (0, j_pad)), constant_values=value)
    xp = xp.at[:, :, -1:, :W:s].set(x[:, :, -1:, ::s])
    xp = xp.at[:, :, :H:s, -1:].set(x[:, :, ::s, -1:])
    xp = xp.at[:, :, -1:, -1:].set(x[:, :, -1:, -1:])
    if i_pad > 0:
        xp = xp.at[:, :, H - 1:H, :W:s].set(value)
    if j_pad > 0:
        xp = xp.at[:, :, :H:s, W - 1:W].set(value)
    if i_pad > 0 and j_pad > 0:
        xp = xp.at[:, :, H - 1:H, W - 1:W].set(value)
    return xp


def _bm_dist_body(xc_ref, xp_ref, out_ref):
    di = pl.program_id(1)
    u = di // 4
    xc = xc_ref[0]

    def dloop(dj, carry):
        vv = dj // 4
        cand = xp_ref[0, dj % 4, pl.ds(u, _NC), pl.ds(vv, _NC), :]
        diff = cand - xc
        d2 = jnp.sum(diff * diff, axis=-1) * (1.0 / 49.0)
        out_ref[0, 0, pl.ds(dj, 1)] = d2[None]
        return carry

    jax.lax.fori_loop(0, _WIN, dloop, 0)


def _bm_distances(xc, xp16):
    N = xc.shape[0]
    dist = pl.pallas_call(
        _bm_dist_body,
        grid=(N, _WIN),
        in_specs=[
            pl.BlockSpec((1, _NC, _NC, 49), lambda n, d: (n, 0, 0, 0)),
            pl.BlockSpec((1, 4, 65, 65, 49), lambda n, d: (n, d % 4, 0, 0, 0)),
        ],
        out_specs=pl.BlockSpec((1, 1, _WIN, _NC, _NC), lambda n, d: (n, d, 0, 0, 0)),
        out_shape=jax.ShapeDtypeStruct((N, _WIN, _WIN, _NC, _NC), jnp.float32),
    )(xc, xp16)
    return dist.reshape(N, _H, _LC)


def _bm_select_body(dist_ref, rr_ref, cc_ref, idx_ref, scratch_ref, *, m):
    iota_h = jax.lax.broadcasted_iota(jnp.int32, (_H, _LC), 0)
    d = dist_ref[0]
    d = jnp.where(iota_h == _SELF, -jnp.inf, d)
    scratch_ref[pl.ds(0, _H), :] = d
    scratch_ref[pl.ds(_H, _HPAD - _H), :] = jnp.full((_HPAD - _H, _LC), jnp.inf, jnp.float32)
    rr = rr_ref[0]
    cc = cc_ref[0]
    iota_s = jax.lax.broadcasted_iota(jnp.int32, (_HPAD, _LC), 0)

    def sel(k, carry):
        dd = scratch_ref[...]
        mn = jnp.min(dd, axis=0, keepdims=True)
        slot = jnp.min(jnp.where(dd == mn, iota_s, 10 ** 6), axis=0)
        rowoff = jnp.floor((slot.astype(jnp.float32) + 0.5) * (1.0 / 37.0)).astype(jnp.int32)
        coloff = slot - rowoff * 37
        row = jnp.minimum(rr + rowoff - _V, _L1D - 1)
        col = jnp.minimum(cc + coloff - _V, _L1D - 1)
        idx_ref[0, pl.ds(k, 1), :] = (row * _L1D + col)[None, :]
        scratch_ref[...] = jnp.where(iota_s == slot[None, :], jnp.inf, dd)
        return carry

    jax.lax.fori_loop(0, m, sel, 0)


def _bm_select(dist, rr, cc, m):
    N = dist.shape[0]
    return pl.pallas_call(
        partial(_bm_select_body, m=m),
        grid=(N,),
        in_specs=[
            pl.BlockSpec((1, _H, _LC), lambda n: (n, 0, 0)),
            pl.BlockSpec((1, _LC), lambda n: (0, 0)),
            pl.BlockSpec((1, _LC), lambda n: (0, 0)),
        ],
        out_specs=pl.BlockSpec((1, m, _LC), lambda n: (n, 0, 0)),
        out_shape=jax.ShapeDtypeStruct((N, m, _LC), jnp.int32),
        scratch_shapes=[pltpu.VMEM((_HPAD, _LC), jnp.float32)],
    )(dist, rr, cc)


def _block_matching(input_x, m, p):
    input_x = jax.lax.stop_gradient(input_x)
    N, C, H, W = input_x.shape
    Hc, Wc = H - p + 1, W - p + 1
    x_patches = _unfold(input_x, p).reshape(N, C * p * p, Hc, Wc)
    x_patches = _align_corners(x_patches, _STEP, value=float('inf'))
    x_center = x_patches[:, :, ::_STEP, ::_STEP]
    xc = jnp.transpose(x_center, (0, 2, 3, 1))
    x_pad = jnp.pad(x_patches, ((0, 0), (0, 0), (_V, _V), (_V, _V)), constant_values=float('inf'))
    x_pad = jnp.pad(x_pad, ((0, 0), (0, 0), (0, 3), (0, 3)), constant_values=float('inf'))
    xp16 = x_pad.reshape(N, 49, 65, 4, 65, 4)
    xp16 = jnp.transpose(xp16, (0, 3, 5, 2, 4, 1)).reshape(N, 16, 65, 65, 49)

    dist = _bm_distances(xc, xp16)

    lane = np.arange(_LC)
    rr = jnp.asarray(np.minimum(4 * (lane // _NC), _L1D - 1), jnp.int32).reshape(1, _LC)
    cc = jnp.asarray(np.minimum(4 * (lane % _NC), _L1D - 1), jnp.int32).reshape(1, _LC)
    idx = _bm_select(dist, rr, cc, m)
    return jnp.transpose(idx, (0, 2, 1)).reshape(N, -1)


def _group_patches(input_y, indices, m, n, p):
    N = input_y.shape[0]
    unfold_y = _unfold(input_y, p)
    Y = jax.vmap(lambda u, ix: u[:, ix])(unfold_y, indices)
    return jnp.transpose(Y, (0, 2, 1)).reshape(N, -1, m, n)


def _denoise_body(g_ref, y_ref, scal_ref, xh_ref, w_ref, *, m, bg):
    G = g_ref[...]
    A = jax.lax.dot_general(G, G, (((2,), (2,)), ((0,), (0,))),
                            preferred_element_type=jnp.float32)
    ii = jax.lax.broadcasted_iota(jnp.int32, (1, m, m), 1)
    jj = jax.lax.broadcasted_iota(jnp.int32, (1, m, m), 2)
    eye = ii == jj
    eps = scal_ref[0, 0]
    A = A + jnp.where(eye, eps, 0.0)

    def gj(k, Ak):
        on_k_row = ii == k
        on_k_col = jj == k
        pivot = jnp.sum(jnp.where(on_k_row & on_k_col, Ak, 0.0), axis=(1, 2), keepdims=True)
        row = jnp.sum(jnp.where(on_k_row, Ak, 0.0), axis=1, keepdims=True)
        col = jnp.sum(jnp.where(on_k_col, Ak, 0.0), axis=2, keepdims=True)
        inv_p = 1.0 / pivot
        base = Ak - col * (row * inv_p)
        return jnp.where(on_k_row & on_k_col, inv_p,
                         jnp.where(on_k_row, row * inv_p,
                                   jnp.where(on_k_col, -col * inv_p, base)))

    Ainv = jax.lax.fori_loop(0, m, gj, A)
    nsig2 = scal_ref[0, 1]
    theta = jnp.where(eye, 1.0, 0.0) - nsig2 * Ainv
    Y = y_ref[...]
    xh_ref[...] = jax.lax.dot_general(theta, Y, (((2,), (1,)), ((0,), (0,))),
                                      preferred_element_type=jnp.float32)
    w = jnp.sum(theta * theta, axis=2)
    w_ref[...] = 1.0 / jnp.clip(w, 1.0 / m, 1.0)


def _denoise(Gsrc, Ymat, eps, nsig2):
    N, B, m, n = Ymat.shape
    NB = N * B
    bg = 64
    G2 = Gsrc.reshape(NB, m, n)
    Y2 = Ymat.reshape(NB, m, n)
    scal = jnp.stack([eps, nsig2]).reshape(1, 2).astype(jnp.float32)
    Xh, W = pl.pallas_call(
        partial(_denoise_body, m=m, bg=bg),
        grid=(NB // bg,),
        in_specs=[
            pl.BlockSpec((bg, m, n), lambda i: (i, 0, 0)),
            pl.BlockSpec((bg, m, n), lambda i: (i, 0, 0)),
            pl.BlockSpec((1, 2), lambda i: (0, 0)),
        ],
        out_specs=[
            pl.BlockSpec((bg, m, n), lambda i: (i, 0, 0)),
            pl.BlockSpec((bg, m), lambda i: (i, 0)),
        ],
        out_shape=[
            jax.ShapeDtypeStruct((NB, m, n), jnp.float32),
            jax.ShapeDtypeStruct((NB, m), jnp.float32),
        ],
    )(G2, Y2, scal)
    return Xh.reshape(N, B, m, n), W.reshape(N, B, m, 1)


def _aggregation(X_hat, weights, indices, size, p):
    N, C, H, W = size
    n = C * p * p
    Ltot = (H - p + 1) * (W - p + 1)
    X_hat = X_hat * weights
    Xp = jnp.transpose(X_hat, (0, 3, 1, 2)).reshape(N, n, -1)
    wts = jnp.broadcast_to(weights.reshape(N, 1, -1), Xp.shape)
    scat = lambda ix, vals: jnp.zeros((n, Ltot), vals.dtype).at[:, ix].add(vals)
    X_sum = jax.vmap(scat)(indices, Xp)
    divisor = jax.vmap(scat)(indices, wts)
    return _pallas_div(_fold(X_sum, H, W, p, C), _fold(divisor, H, W, p, C))


def _div_body(a_ref, b_ref, o_ref):
    o_ref[...] = a_ref[...] / b_ref[...]


def _pallas_div(a, b):
    return pl.pallas_call(
        _div_body,
        out_shape=jax.ShapeDtypeStruct(a.shape, a.dtype),
    )(a, b)


def _step1(input_y, sigma):
    p, m = _P, _M1
    y_block = jnp.mean(input_y, axis=1, keepdims=True)
    indices = _block_matching(y_block, m, p)
    C = input_y.shape[1]
    n = C * p * p
    Y = _group_patches(input_y, indices, m, n, p)
    nsig2 = (n * sigma.reshape(()) ** 2).astype(jnp.float32)
    X_hat, weights = _denoise(Y, Y, jnp.float32(0.0), nsig2)
    return _aggregation(X_hat, weights, indices, input_y.shape, p)


def _step2(input_y, input_x, sigma):
    p, m = _P, _M2
    x_block = jnp.mean(input_x, axis=1, keepdims=True)
    indices = _block_matching(x_block, m, p)
    C = input_y.shape[1]
    n = C * p * p
    Y = _group_patches(input_y, indices, m, n, p)
    X = _group_patches(input_x, indices, m, n, p)
    nsig2 = (n * sigma.reshape(()) ** 2).astype(jnp.float32)
    X_hat, weights = _denoise(X, Y, nsig2, nsig2)
    return _aggregation(X_hat, weights, indices, input_y.shape, p)


def kernel(input_y, sigma):
    den1 = _step1(input_y, sigma)
    den2 = _step2(input_y, den1, sigma)
    return den2


# speedup vs baseline: 10.2755x; 1.1665x over previous
import jax
import jax.numpy as jnp
import numpy as np
from functools import partial
from jax.experimental import pallas as pl
from jax.experimental.pallas import tpu as pltpu
from jax.experimental.pallas import tpu_sc as plsc

_P, _M1, _M2, _WIN, _STEP = 7, 18, 55, 37, 4
_V = _WIN // 2
_H = _WIN_SQ = 37 * 37
_HPAD = 1376
_SELF = _V * _WIN + _V
_L1D = 218
_NC = 56
_LC = _NC * _NC


def _unfold(x, p):
    N, C, H, W = x.shape
    Hc, Wc = H - p + 1, W - p + 1
    cols = jnp.stack([x[:, :, di:di + Hc, dj:dj + Wc] for di in range(p) for dj in range(p)], axis=2)
    return cols.reshape(N, C * p * p, Hc * Wc)


def _fold(X, H, W, p, C):
    N = X.shape[0]
    Hc, Wc = H - p + 1, W - p + 1
    Xr = X.reshape(N, C, p, p, Hc, Wc)
    out = jnp.zeros((N, C, H, W), X.dtype)
    for di in range(p):
        for dj in range(p):
            out = out.at[:, :, di:di + Hc, dj:dj + Wc].add(Xr[:, :, di, dj])
    return out


def _align_corners(x, s, value=0):
    N, C, H, W = x.shape
    if s == 1 or (H % s == 1 and W % s == 1):
        return x
    i_pad, j_pad = (s - H % s + 1) % s, (s - W % s + 1) % s
    xp = jnp.pad(x, ((0, 0), (0, 0), (0, i_pad), (0, j_pad)), constant_values=value)
    xp = xp.at[:, :, -1:, :W:s].set(x[:, :, -1:, ::s])
    xp = xp.at[:, :, :H:s, -1:].set(x[:, :, ::s, -1:])
    xp = xp.at[:, :, -1:, -1:].set(x[:, :, -1:, -1:])
    if i_pad > 0:
        xp = xp.at[:, :, H - 1:H, :W:s].set(value)
    if j_pad > 0:
        xp = xp.at[:, :, :H:s, W - 1:W].set(value)
    if i_pad > 0 and j_pad > 0:
        xp = xp.at[:, :, H - 1:H, W - 1:W].set(value)
    return xp


def _bm_dist_body(xc_ref, xp_ref, out_ref):
    di = pl.program_id(1)
    u = di // 4
    xc = xc_ref[0]

    def dloop(dj, carry):
        vv = dj // 4
        cand = xp_ref[0, dj % 4, pl.ds(u, _NC), pl.ds(vv, _NC), :]
        diff = cand - xc
        d2 = jnp.sum(diff * diff, axis=-1) * (1.0 / 49.0)
        out_ref[0, 0, pl.ds(dj, 1)] = d2[None]
        return carry

    jax.lax.fori_loop(0, _WIN, dloop, 0)


def _bm_distances(xc, xp16):
    N = xc.shape[0]
    dist = pl.pallas_call(
        _bm_dist_body,
        grid=(N, _WIN),
        in_specs=[
            pl.BlockSpec((1, _NC, _NC, 49), lambda n, d: (n, 0, 0, 0)),
            pl.BlockSpec((1, 4, 65, 65, 49), lambda n, d: (n, d % 4, 0, 0, 0)),
        ],
        out_specs=pl.BlockSpec((1, 1, _WIN, _NC, _NC), lambda n, d: (n, d, 0, 0, 0)),
        out_shape=jax.ShapeDtypeStruct((N, _WIN, _WIN, _NC, _NC), jnp.float32),
    )(xc, xp16)
    return dist.reshape(N, _H, _LC)


def _bm_select_body(dist_ref, rr_ref, cc_ref, idx_ref, scratch_ref, *, m):
    iota_h = jax.lax.broadcasted_iota(jnp.int32, (_H, _LC), 0)
    d = dist_ref[0]
    d = jnp.where(iota_h == _SELF, -jnp.inf, d)
    scratch_ref[pl.ds(0, _H), :] = d
    scratch_ref[pl.ds(_H, _HPAD - _H), :] = jnp.full((_HPAD - _H, _LC), jnp.inf, jnp.float32)
    rr = rr_ref[0]
    cc = cc_ref[0]
    iota_s = jax.lax.broadcasted_iota(jnp.int32, (_HPAD, _LC), 0)

    def sel(k, carry):
        dd = scratch_ref[...]
        mn = jnp.min(dd, axis=0, keepdims=True)
        slot = jnp.min(jnp.where(dd == mn, iota_s, 10 ** 6), axis=0)
        rowoff = jnp.floor((slot.astype(jnp.float32) + 0.5) * (1.0 / 37.0)).astype(jnp.int32)
        coloff = slot - rowoff * 37
        row = jnp.minimum(rr + rowoff - _V, _L1D - 1)
        col = jnp.minimum(cc + coloff - _V, _L1D - 1)
        idx_ref[0, pl.ds(k, 1), :] = (row * _L1D + col)[None, :]
        scratch_ref[...] = jnp.where(iota_s == slot[None, :], jnp.inf, dd)
        return carry

    jax.lax.fori_loop(0, m, sel, 0)


def _bm_select(dist, rr, cc, m):
    N = dist.shape[0]
    return pl.pallas_call(
        partial(_bm_select_body, m=m),
        grid=(N,),
        in_specs=[
            pl.BlockSpec((1, _H, _LC), lambda n: (n, 0, 0)),
            pl.BlockSpec((1, _LC), lambda n: (0, 0)),
            pl.BlockSpec((1, _LC), lambda n: (0, 0)),
        ],
        out_specs=pl.BlockSpec((1, m, _LC), lambda n: (n, 0, 0)),
        out_shape=jax.ShapeDtypeStruct((N, m, _LC), jnp.int32),
        scratch_shapes=[pltpu.VMEM((_HPAD, _LC), jnp.float32)],
    )(dist, rr, cc)


def _block_matching(input_x, m, p):
    input_x = jax.lax.stop_gradient(input_x)
    N, C, H, W = input_x.shape
    Hc, Wc = H - p + 1, W - p + 1
    x_patches = _unfold(input_x, p).reshape(N, C * p * p, Hc, Wc)
    x_patches = _align_corners(x_patches, _STEP, value=float('inf'))
    x_center = x_patches[:, :, ::_STEP, ::_STEP]
    xc = jnp.transpose(x_center, (0, 2, 3, 1))
    x_pad = jnp.pad(x_patches, ((0, 0), (0, 0), (_V, _V), (_V, _V)), constant_values=float('inf'))
    x_pad = jnp.pad(x_pad, ((0, 0), (0, 0), (0, 3), (0, 3)), constant_values=float('inf'))
    xp16 = x_pad.reshape(N, 49, 65, 4, 65, 4)
    xp16 = jnp.transpose(xp16, (0, 3, 5, 2, 4, 1)).reshape(N, 16, 65, 65, 49)

    dist = _bm_distances(xc, xp16)

    lane = np.arange(_LC)
    rr = jnp.asarray(np.minimum(4 * (lane // _NC), _L1D - 1), jnp.int32).reshape(1, _LC)
    cc = jnp.asarray(np.minimum(4 * (lane % _NC), _L1D - 1), jnp.int32).reshape(1, _LC)
    idx = _bm_select(dist, rr, cc, m)
    return jnp.transpose(idx, (0, 2, 1)).reshape(N, -1)


_SC_CHUNK = 256
_SC_NW = 32
_DPAD = 256


def _sc_gather(table, idx_flat):
    B = idx_flat.shape[0]
    b_per_w = _SC_CHUNK * (-(-B // (_SC_NW * _SC_CHUNK)))
    B_pad = b_per_w * _SC_NW
    n_chunks = b_per_w // _SC_CHUNK
    idx_pad = jnp.pad(idx_flat, (0, B_pad - B))
    mesh = plsc.VectorSubcoreMesh(core_axis_name="c", subcore_axis_name="s")

    @partial(
        pl.kernel, mesh=mesh,
        out_type=jax.ShapeDtypeStruct((B_pad, _DPAD), jnp.float32),
        scratch_types=[
            pltpu.VMEM((_SC_CHUNK,), jnp.int32),
            pltpu.VMEM((_SC_CHUNK, _DPAD), jnp.float32),
            pltpu.SemaphoreType.DMA,
        ],
    )
    def gather_k(table_hbm, idx_hbm, out_hbm, idx_v, rows_v, sem):
        wid = jax.lax.axis_index("s") * 2 + jax.lax.axis_index("c")

        def body(i, carry):
            base = wid * b_per_w + i * _SC_CHUNK
            pltpu.sync_copy(idx_hbm.at[pl.ds(base, _SC_CHUNK)], idx_v)
            pltpu.async_copy(table_hbm.at[idx_v], rows_v, sem).wait()
            pltpu.sync_copy(rows_v, out_hbm.at[pl.ds(base, _SC_CHUNK)])
            return carry

        jax.lax.fori_loop(0, n_chunks, body, 0)

    return gather_k(table, idx_pad)[:B]


def _patch_table(x, p, n):
    N = x.shape[0]
    u = _unfold(x, p)
    t = jnp.transpose(u, (0, 2, 1)).reshape(-1, n)
    return jnp.pad(t, ((0, 0), (0, _DPAD - n)))


def _flat_indices(indices, Ltot):
    N = indices.shape[0]
    gi = indices + (jnp.arange(N, dtype=indices.dtype) * Ltot)[:, None]
    return gi.reshape(-1)


def _group_patches(input_y, indices, m, n, p):
    N, _, H, W = input_y.shape
    Ltot = (H - p + 1) * (W - p + 1)
    table = _patch_table(input_y, p, n)
    rows = _sc_gather(table, _flat_indices(indices, Ltot))
    return rows[:, :n].reshape(N, -1, m, n)


def _denoise_body(g_ref, y_ref, scal_ref, xh_ref, w_ref, *, m, bg):
    G = g_ref[...]
    A = jax.lax.dot_general(G, G, (((2,), (2,)), ((0,), (0,))),
                            preferred_element_type=jnp.float32)
    ii = jax.lax.broadcasted_iota(jnp.int32, (1, m, m), 1)
    jj = jax.lax.broadcasted_iota(jnp.int32, (1, m, m), 2)
    eye = ii == jj
    eps = scal_ref[0, 0]
    A = A + jnp.where(eye, eps, 0.0)

    def gj(k, Ak):
        on_k_row = ii == k
        on_k_col = jj == k
        pivot = jnp.sum(jnp.where(on_k_row & on_k_col, Ak, 0.0), axis=(1, 2), keepdims=True)
        row = jnp.sum(jnp.where(on_k_row, Ak, 0.0), axis=1, keepdims=True)
        col = jnp.sum(jnp.where(on_k_col, Ak, 0.0), axis=2, keepdims=True)
        inv_p = 1.0 / pivot
        base = Ak - col * (row * inv_p)
        return jnp.where(on_k_row & on_k_col, inv_p,
                         jnp.where(on_k_row, row * inv_p,
                                   jnp.where(on_k_col, -col * inv_p, base)))

    Ainv = jax.lax.fori_loop(0, m, gj, A)
    nsig2 = scal_ref[0, 1]
    theta = jnp.where(eye, 1.0, 0.0) - nsig2 * Ainv
    Y = y_ref[...]
    xh_ref[...] = jax.lax.dot_general(theta, Y, (((2,), (1,)), ((0,), (0,))),
                                      preferred_element_type=jnp.float32)
    w = jnp.sum(theta * theta, axis=2)
    w_ref[...] = 1.0 / jnp.clip(w, 1.0 / m, 1.0)


def _denoise(Gsrc, Ymat, eps, nsig2):
    N, B, m, n = Ymat.shape
    NB = N * B
    bg = 64
    G2 = Gsrc.reshape(NB, m, n)
    Y2 = Ymat.reshape(NB, m, n)
    scal = jnp.stack([eps, nsig2]).reshape(1, 2).astype(jnp.float32)
    Xh, W = pl.pallas_call(
        partial(_denoise_body, m=m, bg=bg),
        grid=(NB // bg,),
        in_specs=[
            pl.BlockSpec((bg, m, n), lambda i: (i, 0, 0)),
            pl.BlockSpec((bg, m, n), lambda i: (i, 0, 0)),
            pl.BlockSpec((1, 2), lambda i: (0, 0)),
        ],
        out_specs=[
            pl.BlockSpec((bg, m, n), lambda i: (i, 0, 0)),
            pl.BlockSpec((bg, m), lambda i: (i, 0)),
        ],
        out_shape=[
            jax.ShapeDtypeStruct((NB, m, n), jnp.float32),
            jax.ShapeDtypeStruct((NB, m), jnp.float32),
        ],
    )(G2, Y2, scal)
    return Xh.reshape(N, B, m, n), W.reshape(N, B, m, 1)


def _aggregation(X_hat, weights, indices, size, p):
    N, C, H, W = size
    n = C * p * p
    Ltot = (H - p + 1) * (W - p + 1)
    X_hat = X_hat * weights
    Xp = jnp.transpose(X_hat, (0, 3, 1, 2)).reshape(N, n, -1)
    wts = jnp.broadcast_to(weights.reshape(N, 1, -1), Xp.shape)
    scat = lambda ix, vals: jnp.zeros((n, Ltot), vals.dtype).at[:, ix].add(vals)
    X_sum = jax.vmap(scat)(indices, Xp)
    divisor = jax.vmap(scat)(indices, wts)
    return _pallas_div(_fold(X_sum, H, W, p, C), _fold(divisor, H, W, p, C))


def _div_body(a_ref, b_ref, o_ref):
    o_ref[...] = a_ref[...] / b_ref[...]


def _pallas_div(a, b):
    return pl.pallas_call(
        _div_body,
        out_shape=jax.ShapeDtypeStruct(a.shape, a.dtype),
    )(a, b)


def _step1(input_y, sigma):
    p, m = _P, _M1
    y_block = jnp.mean(input_y, axis=1, keepdims=True)
    indices = _block_matching(y_block, m, p)
    C = input_y.shape[1]
    n = C * p * p
    Y = _group_patches(input_y, indices, m, n, p)
    nsig2 = (n * sigma.reshape(()) ** 2).astype(jnp.float32)
    X_hat, weights = _denoise(Y, Y, jnp.float32(0.0), nsig2)
    return _aggregation(X_hat, weights, indices, input_y.shape, p)


def _step2(input_y, input_x, sigma):
    p, m = _P, _M2
    x_block = jnp.mean(input_x, axis=1, keepdims=True)
    indices = _block_matching(x_block, m, p)
    C = input_y.shape[1]
    n = C * p * p
    Y = _group_patches(input_y, indices, m, n, p)
    X = _group_patches(input_x, indices, m, n, p)
    nsig2 = (n * sigma.reshape(()) ** 2).astype(jnp.float32)
    X_hat, weights = _denoise(X, Y, nsig2, nsig2)
    return _aggregation(X_hat, weights, indices, input_y.shape, p)


def kernel(input_y, sigma):
    den1 = _step1(input_y, sigma)
    den2 = _step2(input_y, den1, sigma)
    return den2


# speedup vs baseline: 13.7755x; 1.3406x over previous
import jax
import jax.numpy as jnp
import numpy as np
from functools import partial
from jax.experimental import pallas as pl
from jax.experimental.pallas import tpu as pltpu
from jax.experimental.pallas import tpu_sc as plsc

_P, _M1, _M2, _WIN, _STEP = 7, 18, 55, 37, 4
_V = _WIN // 2
_H = _WIN_SQ = 37 * 37
_HPAD = 1376
_SELF = _V * _WIN + _V
_L1D = 218
_NC = 56
_LC = _NC * _NC


def _unfold(x, p):
    N, C, H, W = x.shape
    Hc, Wc = H - p + 1, W - p + 1
    cols = jnp.stack([x[:, :, di:di + Hc, dj:dj + Wc] for di in range(p) for dj in range(p)], axis=2)
    return cols.reshape(N, C * p * p, Hc * Wc)


def _fold(X, H, W, p, C):
    N = X.shape[0]
    Hc, Wc = H - p + 1, W - p + 1
    Xr = X.reshape(N, C, p, p, Hc, Wc)
    out = jnp.zeros((N, C, H, W), X.dtype)
    for di in range(p):
        for dj in range(p):
            out = out.at[:, :, di:di + Hc, dj:dj + Wc].add(Xr[:, :, di, dj])
    return out


def _align_corners(x, s, value=0):
    N, C, H, W = x.shape
    if s == 1 or (H % s == 1 and W % s == 1):
        return x
    i_pad, j_pad = (s - H % s + 1) % s, (s - W % s + 1) % s
    xp = jnp.pad(x, ((0, 0), (0, 0), (0, i_pad), (0, j_pad)), constant_values=value)
    xp = xp.at[:, :, -1:, :W:s].set(x[:, :, -1:, ::s])
    xp = xp.at[:, :, :H:s, -1:].set(x[:, :, ::s, -1:])
    xp = xp.at[:, :, -1:, -1:].set(x[:, :, -1:, -1:])
    if i_pad > 0:
        xp = xp.at[:, :, H - 1:H, :W:s].set(value)
    if j_pad > 0:
        xp = xp.at[:, :, :H:s, W - 1:W].set(value)
    if i_pad > 0 and j_pad > 0:
        xp = xp.at[:, :, H - 1:H, W - 1:W].set(value)
    return xp


def _bm_dist_body(xc_ref, xp_ref, out_ref):
    di = pl.program_id(1)
    u = di // 4
    xc = xc_ref[0]

    def dloop(dj, carry):
        vv = dj // 4
        cand = xp_ref[0, dj % 4, pl.ds(u, _NC), pl.ds(vv, _NC), :]
        diff = cand - xc
        d2 = jnp.sum(diff * diff, axis=-1) * (1.0 / 49.0)
        out_ref[0, 0, pl.ds(dj, 1)] = d2[None]
        return carry

    jax.lax.fori_loop(0, _WIN, dloop, 0)


def _bm_distances(xc, xp16):
    N = xc.shape[0]
    dist = pl.pallas_call(
        _bm_dist_body,
        grid=(N, _WIN),
        in_specs=[
            pl.BlockSpec((1, _NC, _NC, 49), lambda n, d: (n, 0, 0, 0)),
            pl.BlockSpec((1, 4, 65, 65, 49), lambda n, d: (n, d % 4, 0, 0, 0)),
        ],
        out_specs=pl.BlockSpec((1, 1, _WIN, _NC, _NC), lambda n, d: (n, d, 0, 0, 0)),
        out_shape=jax.ShapeDtypeStruct((N, _WIN, _WIN, _NC, _NC), jnp.float32),
    )(xc, xp16)
    return dist.reshape(N, _H, _LC)


def _bm_select_body(dist_ref, rr_ref, cc_ref, idx_ref, scratch_ref, *, m):
    iota_h = jax.lax.broadcasted_iota(jnp.int32, (_H, _LC), 0)
    d = dist_ref[0]
    d = jnp.where(iota_h == _SELF, -jnp.inf, d)
    scratch_ref[pl.ds(0, _H), :] = d
    scratch_ref[pl.ds(_H, _HPAD - _H), :] = jnp.full((_HPAD - _H, _LC), jnp.inf, jnp.float32)
    rr = rr_ref[0]
    cc = cc_ref[0]
    iota_s = jax.lax.broadcasted_iota(jnp.int32, (_HPAD, _LC), 0)

    def sel(k, carry):
        dd = scratch_ref[...]
        mn = jnp.min(dd, axis=0, keepdims=True)
        slot = jnp.min(jnp.where(dd == mn, iota_s, 10 ** 6), axis=0)
        rowoff = jnp.floor((slot.astype(jnp.float32) + 0.5) * (1.0 / 37.0)).astype(jnp.int32)
        coloff = slot - rowoff * 37
        row = jnp.minimum(rr + rowoff - _V, _L1D - 1)
        col = jnp.minimum(cc + coloff - _V, _L1D - 1)
        idx_ref[0, pl.ds(k, 1), :] = (row * _L1D + col)[None, :]
        scratch_ref[...] = jnp.where(iota_s == slot[None, :], jnp.inf, dd)
        return carry

    jax.lax.fori_loop(0, m, sel, 0)


def _bm_select(dist, rr, cc, m):
    N = dist.shape[0]
    return pl.pallas_call(
        partial(_bm_select_body, m=m),
        grid=(N,),
        in_specs=[
            pl.BlockSpec((1, _H, _LC), lambda n: (n, 0, 0)),
            pl.BlockSpec((1, _LC), lambda n: (0, 0)),
            pl.BlockSpec((1, _LC), lambda n: (0, 0)),
        ],
        out_specs=pl.BlockSpec((1, m, _LC), lambda n: (n, 0, 0)),
        out_shape=jax.ShapeDtypeStruct((N, m, _LC), jnp.int32),
        scratch_shapes=[pltpu.VMEM((_HPAD, _LC), jnp.float32)],
    )(dist, rr, cc)


def _block_matching(input_x, m, p):
    input_x = jax.lax.stop_gradient(input_x)
    N, C, H, W = input_x.shape
    Hc, Wc = H - p + 1, W - p + 1
    x_patches = _unfold(input_x, p).reshape(N, C * p * p, Hc, Wc)
    x_patches = _align_corners(x_patches, _STEP, value=float('inf'))
    x_center = x_patches[:, :, ::_STEP, ::_STEP]
    xc = jnp.transpose(x_center, (0, 2, 3, 1))
    x_pad = jnp.pad(x_patches, ((0, 0), (0, 0), (_V, _V), (_V, _V)), constant_values=float('inf'))
    x_pad = jnp.pad(x_pad, ((0, 0), (0, 0), (0, 3), (0, 3)), constant_values=float('inf'))
    xp16 = x_pad.reshape(N, 49, 65, 4, 65, 4)
    xp16 = jnp.transpose(xp16, (0, 3, 5, 2, 4, 1)).reshape(N, 16, 65, 65, 49)

    dist = _bm_distances(xc, xp16)

    lane = np.arange(_LC)
    rr = jnp.asarray(np.minimum(4 * (lane // _NC), _L1D - 1), jnp.int32).reshape(1, _LC)
    cc = jnp.asarray(np.minimum(4 * (lane % _NC), _L1D - 1), jnp.int32).reshape(1, _LC)
    idx = _bm_select(dist, rr, cc, m)
    return jnp.transpose(idx, (0, 2, 1)).reshape(N, -1)


_SC_CHUNK = 256
_SC_NW = 32
_DPAD = 256


def _sc_gather(table, idx_flat):
    B = idx_flat.shape[0]
    b_per_w = _SC_CHUNK * (-(-B // (_SC_NW * _SC_CHUNK)))
    B_pad = b_per_w * _SC_NW
    n_chunks = b_per_w // _SC_CHUNK
    idx_pad = jnp.pad(idx_flat, (0, B_pad - B))
    mesh = plsc.VectorSubcoreMesh(core_axis_name="c", subcore_axis_name="s")

    @partial(
        pl.kernel, mesh=mesh,
        out_type=jax.ShapeDtypeStruct((B_pad, _DPAD), jnp.float32),
        scratch_types=[
            pltpu.VMEM((_SC_CHUNK,), jnp.int32),
            pltpu.VMEM((_SC_CHUNK, _DPAD), jnp.float32),
            pltpu.SemaphoreType.DMA,
        ],
    )
    def gather_k(table_hbm, idx_hbm, out_hbm, idx_v, rows_v, sem):
        wid = jax.lax.axis_index("s") * 2 + jax.lax.axis_index("c")

        def body(i, carry):
            base = wid * b_per_w + i * _SC_CHUNK
            pltpu.sync_copy(idx_hbm.at[pl.ds(base, _SC_CHUNK)], idx_v)
            pltpu.async_copy(table_hbm.at[idx_v], rows_v, sem).wait()
            pltpu.sync_copy(rows_v, out_hbm.at[pl.ds(base, _SC_CHUNK)])
            return carry

        jax.lax.fori_loop(0, n_chunks, body, 0)

    return gather_k(table, idx_pad)[:B]


def _patch_table(x, p, n):
    N = x.shape[0]
    u = _unfold(x, p)
    t = jnp.transpose(u, (0, 2, 1)).reshape(-1, n)
    return jnp.pad(t, ((0, 0), (0, _DPAD - n)))


def _flat_indices(indices, Ltot):
    N = indices.shape[0]
    gi = indices + (jnp.arange(N, dtype=indices.dtype) * Ltot)[:, None]
    return gi.reshape(-1)


def _group_rows(table, idx_flat, N, m, n):
    rows = _sc_gather(table, idx_flat)
    return rows[:, :n].reshape(N, -1, m, n)


def _denoise_body(g_ref, y_ref, scal_ref, xh_ref, w_ref, *, m, bg):
    G = g_ref[...]
    A = jax.lax.dot_general(G, G, (((2,), (2,)), ((0,), (0,))),
                            preferred_element_type=jnp.float32)
    ii = jax.lax.broadcasted_iota(jnp.int32, (1, m, m), 1)
    jj = jax.lax.broadcasted_iota(jnp.int32, (1, m, m), 2)
    eye = ii == jj
    eps = scal_ref[0, 0]
    A = A + jnp.where(eye, eps, 0.0)

    def gj(k, Ak):
        on_k_row = ii == k
        on_k_col = jj == k
        pivot = jnp.sum(jnp.where(on_k_row & on_k_col, Ak, 0.0), axis=(1, 2), keepdims=True)
        row = jnp.sum(jnp.where(on_k_row, Ak, 0.0), axis=1, keepdims=True)
        col = jnp.sum(jnp.where(on_k_col, Ak, 0.0), axis=2, keepdims=True)
        inv_p = 1.0 / pivot
        base = Ak - col * (row * inv_p)
        return jnp.where(on_k_row & on_k_col, inv_p,
                         jnp.where(on_k_row, row * inv_p,
                                   jnp.where(on_k_col, -col * inv_p, base)))

    Ainv = jax.lax.fori_loop(0, m, gj, A)
    nsig2 = scal_ref[0, 1]
    theta = jnp.where(eye, 1.0, 0.0) - nsig2 * Ainv
    Y = y_ref[...]
    w = 1.0 / jnp.clip(jnp.sum(theta * theta, axis=2), 1.0 / m, 1.0)
    w_ref[...] = w
    xh = jax.lax.dot_general(theta, Y, (((2,), (1,)), ((0,), (0,))),
                             preferred_element_type=jnp.float32)
    xh_ref[...] = xh * w[:, :, None]


def _denoise(Gsrc, Ymat, eps, nsig2):
    N, B, m, n = Ymat.shape
    NB = N * B
    bg = 64
    G2 = Gsrc.reshape(NB, m, n)
    Y2 = Ymat.reshape(NB, m, n)
    scal = jnp.stack([eps, nsig2]).reshape(1, 2).astype(jnp.float32)
    Xh, W = pl.pallas_call(
        partial(_denoise_body, m=m, bg=bg),
        grid=(NB // bg,),
        in_specs=[
            pl.BlockSpec((bg, m, n), lambda i: (i, 0, 0)),
            pl.BlockSpec((bg, m, n), lambda i: (i, 0, 0)),
            pl.BlockSpec((1, 2), lambda i: (0, 0)),
        ],
        out_specs=[
            pl.BlockSpec((bg, m, n), lambda i: (i, 0, 0)),
            pl.BlockSpec((bg, m), lambda i: (i, 0)),
        ],
        out_shape=[
            jax.ShapeDtypeStruct((NB, m, n), jnp.float32),
            jax.ShapeDtypeStruct((NB, m), jnp.float32),
        ],
    )(G2, Y2, scal)
    return Xh.reshape(N, B, m, n), W.reshape(N, B, m, 1)


def _aggregation(X_hat_w, weights, indices, size, p):
    N, C, H, W = size
    n = C * p * p
    Ltot = (H - p + 1) * (W - p + 1)
    rows = X_hat_w.reshape(N, -1, n)
    wf = weights.reshape(N, -1)
    scat_rows = lambda ix, vals: jnp.zeros((Ltot, n), vals.dtype).at[ix].add(vals)
    X_sum = jax.vmap(scat_rows)(indices, rows)
    scat_w = lambda ix, vals: jnp.zeros((Ltot,), vals.dtype).at[ix].add(vals)
    div1 = jax.vmap(scat_w)(indices, wf)
    num = _fold(jnp.transpose(X_sum, (0, 2, 1)), H, W, p, C)
    den = _fold(jnp.broadcast_to(div1[:, None, :], (N, p * p, Ltot)), H, W, p, 1)
    return _pallas_div(num, jnp.broadcast_to(den, num.shape))


def _div_body(a_ref, b_ref, o_ref):
    o_ref[...] = a_ref[...] / b_ref[...]


def _pallas_div(a, b):
    return pl.pallas_call(
        _div_body,
        out_shape=jax.ShapeDtypeStruct(a.shape, a.dtype),
    )(a, b)


def kernel(input_y, sigma):
    N, C, H, W = input_y.shape
    p = _P
    n = C * p * p
    Ltot = (H - p + 1) * (W - p + 1)
    table_y = _patch_table(input_y, p, n)
    nsig2 = (n * sigma.reshape(()) ** 2).astype(jnp.float32)

    y_block = jnp.mean(input_y, axis=1, keepdims=True)
    idx1 = _block_matching(y_block, _M1, p)
    Y1 = _group_rows(table_y, _flat_indices(idx1, Ltot), N, _M1, n)
    Xh1, w1 = _denoise(Y1, Y1, jnp.float32(0.0), nsig2)
    den1 = _aggregation(Xh1, w1, idx1, input_y.shape, p)

    x_block = jnp.mean(den1, axis=1, keepdims=True)
    idx2 = _block_matching(x_block, _M2, p)
    f2 = _flat_indices(idx2, Ltot)
    Y2 = _group_rows(table_y, f2, N, _M2, n)
    X2 = _group_rows(_patch_table(den1, p, n), f2, N, _M2, n)
    Xh2, w2 = _denoise(X2, Y2, nsig2, nsig2)
    return _aggregation(Xh2, w2, idx2, input_y.shape, p)
